# Initial kernel scaffold; baseline (speedup 1.0000x reference)
import jax
import jax.numpy as jnp
from jax.experimental import pallas as pl


def kernel(x, edge_index, edge_weight, W1, b1, ln1_w, ln1_b, W2, b2, ln2_w, ln2_b):
    raise NotImplementedError("write your pallas kernel here")


# speedup vs baseline: 16.7634x; 16.7634x over previous
import functools

import jax
import jax.numpy as jnp
from jax import lax
from jax.experimental import pallas as pl
from jax.experimental.pallas import tpu as pltpu
from jax.experimental.pallas import tpu_sc as plsc

NN = 10000
EE = 320000
DIN = 128
DMID = 256
HALF = 128
NPAD = 10240
NC, NS, L = 2, 16, 16

_f32 = jnp.float32
_i32 = jnp.int32

KB = 80
NCHB = (EE // NS) // KB
KD = 80
KN = (EE // (NC * NS)) // KD
KC = 32
NCH = (EE // NS) // KC
RPW = 632
RPW_LAST = NN - 15 * RPW
MBLK = 1000
NBUF = 4


def _bcast_lane(v, lane):
    idx = jnp.full((L, 1), lane, dtype=_i32)
    return lax.gather(
        v, idx,
        lax.GatherDimensionNumbers(offset_dims=(), collapsed_slice_dims=(0,),
                                   start_index_map=(0,)),
        (1,), mode=lax.GatherScatterMode.PROMISE_IN_BOUNDS)


def _precompute_body(dstB_hbm, wB_hbm, srcD_hbm, dstD_hbm, wD_hbm,
                     norm_hbm, dinv2_hbm,
                     deg_sh, dinv_sh, didx2, wvals, fbuf, fbuf2, fbuf3,
                     dinv_vmem, sidxN, didxN, wN, nrmN,
                     semiB0, semiB1, semiB2, semiB3,
                     semsB0, semsB1, semsB2, semsB3,
                     semiD0, semiD1, semiD2, semiD3,
                     semoD0, semoD1, semoD2, semoD3):
    c = lax.axis_index("c")
    s = lax.axis_index("s")
    semiB = [semiB0, semiB1, semiB2, semiB3]
    semsB = [semsB0, semsB1, semsB2, semsB3]
    semiD = [semiD0, semiD1, semiD2, semiD3]
    semoD = [semoD0, semoD1, semoD2, semoD3]

    for i in range(640 // L):
        fbuf[pl.ds(i * L, L)] = jnp.ones((L,), _f32)
    pltpu.sync_copy(fbuf, deg_sh.at[pl.ds(s * 640, 640)])
    plsc.subcore_barrier()

    def b_start_i(u, ci):
        pltpu.async_copy(dstB_hbm.at[s].at[ci], didx2.at[u], semiB[u])
        pltpu.async_copy(wB_hbm.at[s].at[ci], wvals.at[u], semiB[u])

    def b_wait_i(u, ci):
        pltpu.make_async_copy(dstB_hbm.at[s].at[ci], didx2.at[u],
                              semiB[u]).wait()
        pltpu.make_async_copy(wB_hbm.at[s].at[ci], wvals.at[u],
                              semiB[u]).wait()

    def b_drain_s(u):
        pltpu.make_async_copy(wvals.at[u].at[0], deg_sh.at[didx2.at[u].at[0]],
                              semsB[u]).wait()

    b_start_i(0, 0)
    b_start_i(1, 1)

    def deg_round(r, carry):
        for u in range(4):
            ci = r * 4 + u

            @pl.when(ci < NCHB)
            def _(u=u, ci=ci):
                @pl.when(ci + 2 < NCHB)
                def _():
                    @pl.when(ci >= 2)
                    def _():
                        b_drain_s((u + 2) % 4)
                    b_start_i((u + 2) % 4, ci + 2)
                b_wait_i(u, ci)
                pltpu.async_copy(wvals.at[u].at[0],
                                 deg_sh.at[didx2.at[u].at[0]],
                                 semsB[u], add=True)
        return carry
    lax.fori_loop(0, (NCHB + 3) // 4, deg_round, 0)
    for j in (NCHB - 4, NCHB - 3, NCHB - 2, NCHB - 1):
        b_drain_s(j % 4)
    plsc.subcore_barrier()

    base = s * 640
    pltpu.sync_copy(deg_sh.at[pl.ds(base, 640)], fbuf)
    for i in range(640 // L):
        d = fbuf[pl.ds(i * L, L)]
        sq = (d + jnp.float32(1.0)) * jnp.float32(0.5)
        for _ in range(8):
            sq = (sq + d / sq) * jnp.float32(0.5)
        fbuf2[pl.ds(i * L, L)] = jnp.float32(1.0) / sq
        fbuf3[pl.ds(i * L, L)] = jnp.float32(1.0) / d
    pltpu.sync_copy(fbuf2, dinv_sh.at[pl.ds(base, 640)])

    @pl.when(c == 0)
    def _():
        pltpu.sync_copy(fbuf3, dinv2_hbm.at[pl.ds(base, 640)])
    plsc.subcore_barrier()

    wid = c * NS + s
    pltpu.sync_copy(dinv_sh, dinv_vmem)

    def d_start_i(u, j):
        pltpu.async_copy(srcD_hbm.at[wid].at[j], sidxN.at[u], semiD[u])
        pltpu.async_copy(dstD_hbm.at[wid].at[j], didxN.at[u], semiD[u])
        pltpu.async_copy(wD_hbm.at[wid].at[j], wN.at[u], semiD[u])

    def d_wait_i(u, j):
        pltpu.make_async_copy(srcD_hbm.at[wid].at[j], sidxN.at[u],
                              semiD[u]).wait()
        pltpu.make_async_copy(dstD_hbm.at[wid].at[j], didxN.at[u],
                              semiD[u]).wait()
        pltpu.make_async_copy(wD_hbm.at[wid].at[j], wN.at[u],
                              semiD[u]).wait()

    def d_drain_o(u, j):
        pltpu.make_async_copy(nrmN.at[u], norm_hbm.at[wid].at[j],
                              semoD[u]).wait()

    d_start_i(0, 0)
    d_start_i(1, 1)

    def nrm_round(r, carry):
        for u in range(4):
            j = r * 4 + u

            @pl.when(j < KN)
            def _(u=u, j=j):
                @pl.when(j + 2 < KN)
                def _():
                    @pl.when(j >= 2)
                    def _():
                        d_drain_o((u + 2) % 4, j - 2)
                    d_start_i((u + 2) % 4, j + 2)
                d_wait_i(u, j)

                def grp(jj, cc):
                    sv = sidxN[u, 0, pl.ds(jj * L, L)]
                    dv = didxN[u, 0, pl.ds(jj * L, L)]
                    wv = wN[u, 0, pl.ds(jj * L, L)]
                    a = plsc.load_gather(dinv_vmem, [sv])
                    b = plsc.load_gather(dinv_vmem, [dv])
                    nrmN[u, 0, pl.ds(jj * L, L)] = a * wv * b
                    return cc
                lax.fori_loop(0, KD // L, grp, 0)
                pltpu.async_copy(nrmN.at[u], norm_hbm.at[wid].at[j],
                                 semoD[u])
        return carry
    lax.fori_loop(0, (KN + 3) // 4, nrm_round, 0)
    for j in (KN - 4, KN - 3, KN - 2, KN - 1):
        d_drain_o(j % 4, j)


def _sc_precompute(dstB, wB, srcD, dstD, wD):
    mesh = plsc.VectorSubcoreMesh(core_axis_name="c", subcore_axis_name="s",
                                  num_cores=NC, num_subcores=NS)
    kfn = pl.kernel(
        _precompute_body,
        out_type=(jax.ShapeDtypeStruct((NC * NS, KN, 1, KD), _f32),
                  jax.ShapeDtypeStruct((NPAD,), _f32)),
        mesh=mesh,
        compiler_params=pltpu.CompilerParams(needs_layout_passes=False),
        scratch_types=[
            pltpu.VMEM_SHARED((NPAD,), _f32),
            pltpu.VMEM_SHARED((NPAD,), _f32),
            pltpu.VMEM((4, 1, KB), _i32),
            pltpu.VMEM((4, 1, KB), _f32),
            pltpu.VMEM((640,), _f32),
            pltpu.VMEM((640,), _f32),
            pltpu.VMEM((640,), _f32),
            pltpu.VMEM((NPAD,), _f32),
            pltpu.VMEM((4, 1, KD), _i32),
            pltpu.VMEM((4, 1, KD), _i32),
            pltpu.VMEM((4, 1, KD), _f32),
            pltpu.VMEM((4, 1, KD), _f32),
        ] + [pltpu.SemaphoreType.DMA] * 16 + [
        ],
    )
    return kfn(dstB, wB, srcD, dstD, wD)


def _agg_one_core(h_half, agg_half, src3_hbm, dst3_hbm, norm3_hbm,
                  acc_sh, sidxS, didxS, nrmS, rows, zbuf, semi, semg, sems,
                  s):
    for r in range(8):
        for g in range(HALF // L):
            zbuf[r, pl.ds(g * L, L)] = jnp.zeros((L,), _f32)

    def zrow(k, carry):
        pltpu.sync_copy(zbuf, acc_sh.at[pl.ds(s * RPW + k * 8, 8)])
        return carry

    @pl.when(s < 15)
    def _():
        lax.fori_loop(0, RPW // 8, zrow, 0)

    @pl.when(s == 15)
    def _():
        lax.fori_loop(0, RPW_LAST // 8, zrow, 0)
    plsc.subcore_barrier()

    def start_i(ui, ci):
        pltpu.async_copy(src3_hbm.at[s].at[ci], sidxS.at[ui], semi[ui])
        pltpu.async_copy(dst3_hbm.at[s].at[ci], didxS.at[ui], semi[ui])

    def start_g(ug, ui, ci):
        pltpu.make_async_copy(src3_hbm.at[s].at[ci], sidxS.at[ui],
                              semi[ui]).wait()
        pltpu.make_async_copy(dst3_hbm.at[s].at[ci], didxS.at[ui],
                              semi[ui]).wait()
        pltpu.async_copy(h_half.at[sidxS.at[ui].at[0]], rows[ug], semg[ug])
        pltpu.async_copy(norm3_hbm.at[s].at[ci], nrmS[ug], semg[ug])

    def wait_g(ug, ui, ci):
        pltpu.make_async_copy(h_half.at[sidxS.at[ui].at[0]], rows[ug],
                              semg[ug]).wait()
        pltpu.make_async_copy(norm3_hbm.at[s].at[ci], nrmS[ug],
                              semg[ug]).wait()

    def start_s(ug, ui):
        pltpu.async_copy(rows[ug], acc_sh.at[didxS.at[ui].at[0]], sems[ug],
                         add=True)

    def drain_s(ug, ui):
        pltpu.make_async_copy(rows[ug], acc_sh.at[didxS.at[ui].at[0]],
                              sems[ug]).wait()

    def scale(ug):
        def grp(jj, cc):
            nv = nrmS[ug][0, pl.ds(jj * L, L)]
            for lane in range(L):
                bl = _bcast_lane(nv, lane)
                j = jj * L + lane
                for g in range(HALF // L):
                    rows[ug][j, pl.ds(g * L, L)] = (
                        rows[ug][j, pl.ds(g * L, L)] * bl)
            return cc
        lax.fori_loop(0, KC // L, grp, 0)

    for j in range(4):
        start_i(j, j)
    start_g(0, 0, 0)
    start_g(1, 1, 1)

    def round_body(r, carry):
        for u in range(8):
            ci = r * 8 + u

            @pl.when(ci < NCH)
            def _(u=u, ci=ci):
                @pl.when(ci >= 2)
                def _():
                    drain_s((u - 2) % 4, (u - 2) % 8)

                @pl.when(ci + 2 < NCH)
                def _():
                    start_g((u + 2) % 4, (u + 2) % 8, ci + 2)

                @pl.when(ci + 4 < NCH)
                def _():
                    start_i((u + 4) % 8, ci + 4)

                wait_g(u % 4, u % 8, ci)
                scale(u % 4)
                start_s(u % 4, u % 8)
        return carry
    lax.fori_loop(0, (NCH + 7) // 8 + 1, round_body, 0)

    for j in (NCH - 2, NCH - 1):
        drain_s(j % 4, j % 8)
    plsc.subcore_barrier()

    @pl.when(s < 15)
    def _():
        pltpu.sync_copy(acc_sh.at[pl.ds(s * RPW, RPW)],
                        agg_half.at[pl.ds(s * RPW, RPW)])

    @pl.when(s == 15)
    def _():
        pltpu.sync_copy(acc_sh.at[pl.ds(15 * RPW, RPW_LAST)],
                        agg_half.at[pl.ds(15 * RPW, RPW_LAST)])


def _agg_body(h_hbm, src3_hbm, dst3_hbm, norm3_hbm, agg_hbm,
              acc_sh, sidxS, didxS,
              nrmS0, nrmS1, nrmS2, nrmS3,
              rows0, rows1, rows2, rows3, zbuf,
              semi0, semi1, semi2, semi3, semi4, semi5, semi6, semi7,
              semg0, semg1, semg2, semg3,
              sems0, sems1, sems2, sems3):
    c = lax.axis_index("c")
    s = lax.axis_index("s")
    nrmS = [nrmS0, nrmS1, nrmS2, nrmS3]
    rows = [rows0, rows1, rows2, rows3]
    semi = [semi0, semi1, semi2, semi3, semi4, semi5, semi6, semi7]
    semg = [semg0, semg1, semg2, semg3]
    sems = [sems0, sems1, sems2, sems3]

    @pl.when(c == 0)
    def _():
        _agg_one_core(h_hbm.at[0], agg_hbm.at[0],
                      src3_hbm, dst3_hbm, norm3_hbm,
                      acc_sh, sidxS, didxS, nrmS, rows, zbuf,
                      semi, semg, sems, s)

    @pl.when(c == 1)
    def _():
        _agg_one_core(h_hbm.at[1], agg_hbm.at[1],
                      src3_hbm, dst3_hbm, norm3_hbm,
                      acc_sh, sidxS, didxS, nrmS, rows, zbuf,
                      semi, semg, sems, s)


def _sc_aggregate(h_split, src3, dst3, norm3):
    mesh = plsc.VectorSubcoreMesh(core_axis_name="c", subcore_axis_name="s",
                                  num_cores=NC, num_subcores=NS)
    kfn = pl.kernel(
        _agg_body,
        out_type=jax.ShapeDtypeStruct((NC, NN, HALF), _f32),
        mesh=mesh,
        compiler_params=pltpu.CompilerParams(needs_layout_passes=False),
        scratch_types=[
            pltpu.VMEM_SHARED((NN, HALF), _f32),
            pltpu.VMEM((8, 1, KC), _i32),
            pltpu.VMEM((8, 1, KC), _i32),
            pltpu.VMEM((1, KC), _f32),
            pltpu.VMEM((1, KC), _f32),
            pltpu.VMEM((1, KC), _f32),
            pltpu.VMEM((1, KC), _f32),
            pltpu.VMEM((KC, HALF), _f32),
            pltpu.VMEM((KC, HALF), _f32),
            pltpu.VMEM((KC, HALF), _f32),
            pltpu.VMEM((KC, HALF), _f32),
            pltpu.VMEM((8, HALF), _f32),
        ] + [pltpu.SemaphoreType.DMA] * 16 + [
        ],
    )
    return kfn(h_split, src3, dst3, norm3)


def _linear_body(x_ref, w_ref, h_ref, *, ns):
    acc = jnp.zeros((MBLK, HALF), _f32)
    for si in range(ns):
        acc = acc + jnp.dot(x_ref[si], w_ref[si],
                            preferred_element_type=_f32,
                            precision=lax.Precision.HIGHEST)
    h_ref[0] = acc


def _tc_linear(xs, wr, ns):
    grid = (NN // MBLK, NC)
    return pl.pallas_call(
        functools.partial(_linear_body, ns=ns),
        grid=grid,
        in_specs=[
            pl.BlockSpec((ns, MBLK, DIN), lambda m, cc: (0, m, 0)),
            pl.BlockSpec((ns, DIN, HALF), lambda m, cc: (0, 0, cc)),
        ],
        out_specs=pl.BlockSpec((1, MBLK, HALF), lambda m, cc: (cc, m, 0)),
        out_shape=jax.ShapeDtypeStruct((NC, NN, HALF), _f32),
    )(xs, wr)


def _stats_body(agg_ref, h_ref, d2_ref, b_ref, sum_ref, ssq_ref):
    m = pl.program_id(0)
    a = agg_ref[...] + h_ref[...] * d2_ref[...] + b_ref[...]
    ps = jnp.sum(a)
    pss = jnp.sum(a * a)

    @pl.when(m == 0)
    def _():
        sum_ref[0, 0] = ps
        ssq_ref[0, 0] = pss

    @pl.when(m > 0)
    def _():
        sum_ref[0, 0] = sum_ref[0, 0] + ps
        ssq_ref[0, 0] = ssq_ref[0, 0] + pss


def _tc_stats(agg, h, d2col, b3):
    return pl.pallas_call(
        _stats_body,
        grid=(NN // MBLK,),
        in_specs=[
            pl.BlockSpec((NC, MBLK, HALF), lambda m: (0, m, 0)),
            pl.BlockSpec((NC, MBLK, HALF), lambda m: (0, m, 0)),
            pl.BlockSpec((MBLK, 1), lambda m: (m, 0)),
            pl.BlockSpec((NC, 1, HALF), lambda m: (0, 0, 0)),
        ],
        out_specs=(pl.BlockSpec(memory_space=pltpu.SMEM),
                   pl.BlockSpec(memory_space=pltpu.SMEM)),
        out_shape=(jax.ShapeDtypeStruct((1, 1), _f32),
                   jax.ShapeDtypeStruct((1, 1), _f32)),
    )(agg, h, d2col, b3)


def _norm_mm_body(sum_ref, ssq_ref, agg_ref, h_ref, d2_ref, b_ref, w_ref,
                  lb_ref, w2_ref, o_ref):
    cnt = jnp.float32(NN * DMID)
    mu = sum_ref[0, 0] / cnt
    var = ssq_ref[0, 0] / cnt - mu * mu
    a = agg_ref[...] + h_ref[...] * d2_ref[...] + b_ref[...]
    xn = (a - mu) * lax.rsqrt(var + jnp.float32(1e-5))
    y = jnp.maximum(xn * w_ref[...] + lb_ref[...], 0.0)
    w2 = w2_ref[...]
    acc = (jnp.dot(y[0], w2[0], preferred_element_type=_f32,
                   precision=lax.Precision.HIGHEST)
           + jnp.dot(y[1], w2[1], preferred_element_type=_f32,
                     precision=lax.Precision.HIGHEST))
    o_ref[0] = acc


def _tc_ln_mm(agg, h, d2col, b3, lnw, lnb, w2r):
    ssum, ssq = _tc_stats(agg, h, d2col, b3)
    return pl.pallas_call(
        _norm_mm_body,
        grid=(NN // MBLK, NC),
        in_specs=[
            pl.BlockSpec(memory_space=pltpu.SMEM),
            pl.BlockSpec(memory_space=pltpu.SMEM),
            pl.BlockSpec((NC, MBLK, HALF), lambda m, cc: (0, m, 0)),
            pl.BlockSpec((NC, MBLK, HALF), lambda m, cc: (0, m, 0)),
            pl.BlockSpec((MBLK, 1), lambda m, cc: (m, 0)),
            pl.BlockSpec((NC, 1, HALF), lambda m, cc: (0, 0, 0)),
            pl.BlockSpec((NC, 1, HALF), lambda m, cc: (0, 0, 0)),
            pl.BlockSpec((NC, 1, HALF), lambda m, cc: (0, 0, 0)),
            pl.BlockSpec((NC, HALF, DMID // 2), lambda m, cc: (0, 0, cc)),
        ],
        out_specs=pl.BlockSpec((1, MBLK, HALF), lambda m, cc: (cc, m, 0)),
        out_shape=jax.ShapeDtypeStruct((NC, NN, HALF), _f32),
    )(ssum, ssq, agg, h, d2col, b3, lnw, lnb, w2r)


def _ln_body_merge(agg_ref, h_ref, d2_ref, b_ref, w_ref, lb_ref, o_ref):
    a = agg_ref[...] + h_ref[...] * d2_ref[...] + b_ref[...]
    mu = jnp.mean(a)
    var = jnp.mean((a - mu) ** 2)
    xn = (a - mu) * lax.rsqrt(var + jnp.float32(1e-5))
    y = jnp.maximum(xn * w_ref[...] + lb_ref[...], 0.0)
    o_ref[:, 0:HALF] = y[0]
    o_ref[:, HALF:DMID] = y[1]


def _tc_ln(agg, h, d2col, b3, lnw, lnb):
    return pl.pallas_call(
        _ln_body_merge,
        out_shape=jax.ShapeDtypeStruct((NN, DMID), _f32),
    )(agg, h, d2col, b3, lnw, lnb)


def kernel(x, edge_index, edge_weight, W1, b1, ln1_w, ln1_b, W2, b2, ln2_w, ln2_b):
    src = edge_index[0].astype(_i32)
    dst = edge_index[1].astype(_i32)
    w = edge_weight.astype(_f32)

    dstB = dst.reshape(NS, NCHB, 1, KB)
    wB = w.reshape(NS, NCHB, 1, KB)
    srcD = src.reshape(NC * NS, KN, 1, KD)
    dstD = dst.reshape(NC * NS, KN, 1, KD)
    wD = w.reshape(NC * NS, KN, 1, KD)

    norm3w, dinv2 = _sc_precompute(dstB, wB, srcD, dstD, wD)
    norm3 = norm3w.reshape(NS, NCH, 1, KC)
    src3 = src.reshape(NS, NCH, 1, KC)
    dst3 = dst.reshape(NS, NCH, 1, KC)
    d2col = dinv2[:NN].reshape(NN, 1)

    xs1 = x.reshape(1, NN, DIN)
    w1r = W1.reshape(1, DIN, DMID)
    h1 = _tc_linear(xs1, w1r, ns=1)
    agg1 = _sc_aggregate(h1, src3, dst3, norm3)
    w2r = W2.reshape(NC, HALF, DMID)
    h2 = _tc_ln_mm(agg1, h1, d2col, b1.reshape(NC, 1, HALF),
                   ln1_w.reshape(NC, 1, HALF), ln1_b.reshape(NC, 1, HALF),
                   w2r)
    agg2 = _sc_aggregate(h2, src3, dst3, norm3)
    out = _tc_ln(agg2, h2, d2col, b2.reshape(NC, 1, HALF),
                 ln2_w.reshape(NC, 1, HALF), ln2_b.reshape(NC, 1, HALF))
    return out


# speedup vs baseline: 18.6589x; 1.1131x over previous
import functools

import jax
import jax.numpy as jnp
from jax import lax
from jax.experimental import pallas as pl
from jax.experimental.pallas import tpu as pltpu
from jax.experimental.pallas import tpu_sc as plsc

NN = 10000
EE = 320000
DIN = 128
DMID = 256
HALF = 128
NPAD = 10240
NC, NS, L = 2, 16, 16

_f32 = jnp.float32
_i32 = jnp.int32

KB = 80
NCHB = (EE // NS) // KB
KD = 80
KN = (EE // (NC * NS)) // KD
KC = 32
NCH = (EE // NS) // KC
RPW = 632
RPW_LAST = NN - 15 * RPW
MBLK = 1000
NBUF = 4


def _bcast_lane(v, lane):
    idx = jnp.full((L, 1), lane, dtype=_i32)
    return lax.gather(
        v, idx,
        lax.GatherDimensionNumbers(offset_dims=(), collapsed_slice_dims=(0,),
                                   start_index_map=(0,)),
        (1,), mode=lax.GatherScatterMode.PROMISE_IN_BOUNDS)


def _precompute_body(dst_hbm, w_hbm, src_hbm,
                     norm_hbm, dinv2_hbm,
                     deg_sh, dinv_sh, didx2, wvals, fbuf, fbuf2, fbuf3,
                     dinv_vmem, sidxN, didxN, wN, nrmN,
                     semiB0, semiB1, semiB2, semiB3,
                     semsB0, semsB1, semsB2, semsB3,
                     semiD0, semiD1, semiD2, semiD3,
                     semoD0, semoD1, semoD2, semoD3):
    c = lax.axis_index("c")
    s = lax.axis_index("s")
    semiB = [semiB0, semiB1, semiB2, semiB3]
    semsB = [semsB0, semsB1, semsB2, semsB3]
    semiD = [semiD0, semiD1, semiD2, semiD3]
    semoD = [semoD0, semoD1, semoD2, semoD3]

    for i in range(640 // L):
        fbuf[pl.ds(i * L, L)] = jnp.ones((L,), _f32)
    pltpu.sync_copy(fbuf, deg_sh.at[pl.ds(s * 640, 640)])
    plsc.subcore_barrier()

    def b_start_i(u, ci):
        base = s * (EE // NS) + ci * KB
        pltpu.async_copy(dst_hbm.at[pl.ds(base, KB)], didx2.at[u].at[0],
                         semiB[u])
        pltpu.async_copy(w_hbm.at[pl.ds(base, KB)], wvals.at[u].at[0],
                         semiB[u])

    def b_wait_i(u, ci):
        base = s * (EE // NS) + ci * KB
        pltpu.make_async_copy(dst_hbm.at[pl.ds(base, KB)], didx2.at[u].at[0],
                              semiB[u]).wait()
        pltpu.make_async_copy(w_hbm.at[pl.ds(base, KB)], wvals.at[u].at[0],
                              semiB[u]).wait()

    def b_drain_s(u):
        pltpu.make_async_copy(wvals.at[u].at[0], deg_sh.at[didx2.at[u].at[0]],
                              semsB[u]).wait()

    b_start_i(0, 0)
    b_start_i(1, 1)

    def deg_round(r, carry):
        for u in range(4):
            ci = r * 4 + u

            @pl.when(ci < NCHB)
            def _(u=u, ci=ci):
                @pl.when(ci + 2 < NCHB)
                def _():
                    @pl.when(ci >= 2)
                    def _():
                        b_drain_s((u + 2) % 4)
                    b_start_i((u + 2) % 4, ci + 2)
                b_wait_i(u, ci)
                pltpu.async_copy(wvals.at[u].at[0],
                                 deg_sh.at[didx2.at[u].at[0]],
                                 semsB[u], add=True)
        return carry
    lax.fori_loop(0, (NCHB + 3) // 4, deg_round, 0)
    for j in (NCHB - 4, NCHB - 3, NCHB - 2, NCHB - 1):
        b_drain_s(j % 4)
    plsc.subcore_barrier()

    base = s * 640
    pltpu.sync_copy(deg_sh.at[pl.ds(base, 640)], fbuf)
    for i in range(640 // L):
        d = fbuf[pl.ds(i * L, L)]
        sq = (d + jnp.float32(1.0)) * jnp.float32(0.5)
        for _ in range(8):
            sq = (sq + d / sq) * jnp.float32(0.5)
        fbuf2[pl.ds(i * L, L)] = jnp.float32(1.0) / sq
        fbuf3[pl.ds(i * L, L)] = jnp.float32(1.0) / d
    pltpu.sync_copy(fbuf2, dinv_sh.at[pl.ds(base, 640)])

    @pl.when(c == 0)
    def _():
        pltpu.sync_copy(fbuf3, dinv2_hbm.at[pl.ds(base, 640)])
    plsc.subcore_barrier()

    wid = c * NS + s
    pltpu.sync_copy(dinv_sh, dinv_vmem)

    def d_start_i(u, j):
        base = wid * (EE // (NC * NS)) + j * KD
        pltpu.async_copy(src_hbm.at[pl.ds(base, KD)], sidxN.at[u].at[0],
                         semiD[u])
        pltpu.async_copy(dst_hbm.at[pl.ds(base, KD)], didxN.at[u].at[0],
                         semiD[u])
        pltpu.async_copy(w_hbm.at[pl.ds(base, KD)], wN.at[u].at[0], semiD[u])

    def d_wait_i(u, j):
        base = wid * (EE // (NC * NS)) + j * KD
        pltpu.make_async_copy(src_hbm.at[pl.ds(base, KD)], sidxN.at[u].at[0],
                              semiD[u]).wait()
        pltpu.make_async_copy(dst_hbm.at[pl.ds(base, KD)], didxN.at[u].at[0],
                              semiD[u]).wait()
        pltpu.make_async_copy(w_hbm.at[pl.ds(base, KD)], wN.at[u].at[0],
                              semiD[u]).wait()

    def d_drain_o(u, j):
        base = wid * (EE // (NC * NS)) + j * KD
        pltpu.make_async_copy(nrmN.at[u].at[0],
                              norm_hbm.at[pl.ds(base, KD)], semoD[u]).wait()

    d_start_i(0, 0)
    d_start_i(1, 1)

    def nrm_round(r, carry):
        for u in range(4):
            j = r * 4 + u

            @pl.when(j < KN)
            def _(u=u, j=j):
                @pl.when(j + 2 < KN)
                def _():
                    @pl.when(j >= 2)
                    def _():
                        d_drain_o((u + 2) % 4, j - 2)
                    d_start_i((u + 2) % 4, j + 2)
                d_wait_i(u, j)

                def grp(jj, cc):
                    sv = sidxN[u, 0, pl.ds(jj * L, L)]
                    dv = didxN[u, 0, pl.ds(jj * L, L)]
                    wv = wN[u, 0, pl.ds(jj * L, L)]
                    a = plsc.load_gather(dinv_vmem, [sv])
                    b = plsc.load_gather(dinv_vmem, [dv])
                    nrmN[u, 0, pl.ds(jj * L, L)] = a * wv * b
                    return cc
                lax.fori_loop(0, KD // L, grp, 0)
                base = wid * (EE // (NC * NS)) + j * KD
                pltpu.async_copy(nrmN.at[u].at[0],
                                 norm_hbm.at[pl.ds(base, KD)], semoD[u])
        return carry
    lax.fori_loop(0, (KN + 3) // 4, nrm_round, 0)
    for j in (KN - 4, KN - 3, KN - 2, KN - 1):
        d_drain_o(j % 4, j)


def _sc_precompute(dst, w, srcf):
    mesh = plsc.VectorSubcoreMesh(core_axis_name="c", subcore_axis_name="s",
                                  num_cores=NC, num_subcores=NS)
    kfn = pl.kernel(
        _precompute_body,
        out_type=(jax.ShapeDtypeStruct((EE,), _f32),
                  jax.ShapeDtypeStruct((NPAD,), _f32)),
        mesh=mesh,
        compiler_params=pltpu.CompilerParams(needs_layout_passes=False),
        scratch_types=[
            pltpu.VMEM_SHARED((NPAD,), _f32),
            pltpu.VMEM_SHARED((NPAD,), _f32),
            pltpu.VMEM((4, 1, KB), _i32),
            pltpu.VMEM((4, 1, KB), _f32),
            pltpu.VMEM((640,), _f32),
            pltpu.VMEM((640,), _f32),
            pltpu.VMEM((640,), _f32),
            pltpu.VMEM((NPAD,), _f32),
            pltpu.VMEM((4, 1, KD), _i32),
            pltpu.VMEM((4, 1, KD), _i32),
            pltpu.VMEM((4, 1, KD), _f32),
            pltpu.VMEM((4, 1, KD), _f32),
        ] + [pltpu.SemaphoreType.DMA] * 16 + [
        ],
    )
    return kfn(dst, w, srcf)


def _agg_one_core(h_half, agg_half, src_hbm, dst_hbm, norm_hbm,
                  acc_sh, sidxS, didxS, nrmS, rows, zbuf, semi, semg, sems,
                  s):
    for r in range(8):
        for g in range(HALF // L):
            zbuf[r, pl.ds(g * L, L)] = jnp.zeros((L,), _f32)

    def zrow(k, carry):
        pltpu.sync_copy(zbuf, acc_sh.at[pl.ds(s * RPW + k * 8, 8)])
        return carry

    @pl.when(s < 15)
    def _():
        lax.fori_loop(0, RPW // 8, zrow, 0)

    @pl.when(s == 15)
    def _():
        lax.fori_loop(0, RPW_LAST // 8, zrow, 0)
    plsc.subcore_barrier()

    def start_i(ui, ci):
        base = s * (EE // NS) + ci * KC
        pltpu.async_copy(src_hbm.at[pl.ds(base, KC)], sidxS.at[ui].at[0],
                         semi[ui])
        pltpu.async_copy(dst_hbm.at[pl.ds(base, KC)], didxS.at[ui].at[0],
                         semi[ui])

    def start_g(ug, ui, ci):
        base = s * (EE // NS) + ci * KC
        pltpu.make_async_copy(src_hbm.at[pl.ds(base, KC)],
                              sidxS.at[ui].at[0], semi[ui]).wait()
        pltpu.make_async_copy(dst_hbm.at[pl.ds(base, KC)],
                              didxS.at[ui].at[0], semi[ui]).wait()
        pltpu.async_copy(h_half.at[sidxS.at[ui].at[0]], rows[ug], semg[ug])
        pltpu.async_copy(norm_hbm.at[pl.ds(base, KC)], nrmS[ug].at[0],
                         semg[ug])

    def wait_g(ug, ui, ci):
        base = s * (EE // NS) + ci * KC
        pltpu.make_async_copy(h_half.at[sidxS.at[ui].at[0]], rows[ug],
                              semg[ug]).wait()
        pltpu.make_async_copy(norm_hbm.at[pl.ds(base, KC)], nrmS[ug].at[0],
                              semg[ug]).wait()

    def start_s(ug, ui):
        pltpu.async_copy(rows[ug], acc_sh.at[didxS.at[ui].at[0]], sems[ug],
                         add=True)

    def drain_s(ug, ui):
        pltpu.make_async_copy(rows[ug], acc_sh.at[didxS.at[ui].at[0]],
                              sems[ug]).wait()

    def scale(ug):
        def grp(jj, cc):
            nv = nrmS[ug][0, pl.ds(jj * L, L)]
            for lane in range(L):
                bl = _bcast_lane(nv, lane)
                j = jj * L + lane
                for g in range(HALF // L):
                    rows[ug][j, pl.ds(g * L, L)] = (
                        rows[ug][j, pl.ds(g * L, L)] * bl)
            return cc
        lax.fori_loop(0, KC // L, grp, 0)

    for j in range(4):
        start_i(j, j)
    start_g(0, 0, 0)
    start_g(1, 1, 1)

    def round_body(r, carry):
        for u in range(8):
            ci = r * 8 + u

            @pl.when(ci < NCH)
            def _(u=u, ci=ci):
                @pl.when(ci >= 2)
                def _():
                    drain_s((u - 2) % 4, (u - 2) % 8)

                @pl.when(ci + 2 < NCH)
                def _():
                    start_g((u + 2) % 4, (u + 2) % 8, ci + 2)

                @pl.when(ci + 4 < NCH)
                def _():
                    start_i((u + 4) % 8, ci + 4)

                wait_g(u % 4, u % 8, ci)
                scale(u % 4)
                start_s(u % 4, u % 8)
        return carry
    lax.fori_loop(0, (NCH + 7) // 8 + 1, round_body, 0)

    for j in (NCH - 2, NCH - 1):
        drain_s(j % 4, j % 8)
    plsc.subcore_barrier()

    @pl.when(s < 15)
    def _():
        pltpu.sync_copy(acc_sh.at[pl.ds(s * RPW, RPW)],
                        agg_half.at[pl.ds(s * RPW, RPW)])

    @pl.when(s == 15)
    def _():
        pltpu.sync_copy(acc_sh.at[pl.ds(15 * RPW, RPW_LAST)],
                        agg_half.at[pl.ds(15 * RPW, RPW_LAST)])


def _agg_body(h_hbm, src_hbm, dst_hbm, norm_hbm, agg_hbm,
              acc_sh, sidxS, didxS,
              nrmS0, nrmS1, nrmS2, nrmS3,
              rows0, rows1, rows2, rows3, zbuf,
              semi0, semi1, semi2, semi3, semi4, semi5, semi6, semi7,
              semg0, semg1, semg2, semg3,
              sems0, sems1, sems2, sems3):
    c = lax.axis_index("c")
    s = lax.axis_index("s")
    nrmS = [nrmS0, nrmS1, nrmS2, nrmS3]
    rows = [rows0, rows1, rows2, rows3]
    semi = [semi0, semi1, semi2, semi3, semi4, semi5, semi6, semi7]
    semg = [semg0, semg1, semg2, semg3]
    sems = [sems0, sems1, sems2, sems3]

    @pl.when(c == 0)
    def _():
        _agg_one_core(h_hbm.at[0], agg_hbm.at[0],
                      src_hbm, dst_hbm, norm_hbm,
                      acc_sh, sidxS, didxS, nrmS, rows, zbuf,
                      semi, semg, sems, s)

    @pl.when(c == 1)
    def _():
        _agg_one_core(h_hbm.at[1], agg_hbm.at[1],
                      src_hbm, dst_hbm, norm_hbm,
                      acc_sh, sidxS, didxS, nrmS, rows, zbuf,
                      semi, semg, sems, s)


def _sc_aggregate(h_split, srcf, dstf, normf):
    mesh = plsc.VectorSubcoreMesh(core_axis_name="c", subcore_axis_name="s",
                                  num_cores=NC, num_subcores=NS)
    kfn = pl.kernel(
        _agg_body,
        out_type=jax.ShapeDtypeStruct((NC, NN, HALF), _f32),
        mesh=mesh,
        compiler_params=pltpu.CompilerParams(needs_layout_passes=False),
        scratch_types=[
            pltpu.VMEM_SHARED((NN, HALF), _f32),
            pltpu.VMEM((8, 1, KC), _i32),
            pltpu.VMEM((8, 1, KC), _i32),
            pltpu.VMEM((1, KC), _f32),
            pltpu.VMEM((1, KC), _f32),
            pltpu.VMEM((1, KC), _f32),
            pltpu.VMEM((1, KC), _f32),
            pltpu.VMEM((KC, HALF), _f32),
            pltpu.VMEM((KC, HALF), _f32),
            pltpu.VMEM((KC, HALF), _f32),
            pltpu.VMEM((KC, HALF), _f32),
            pltpu.VMEM((8, HALF), _f32),
        ] + [pltpu.SemaphoreType.DMA] * 16 + [
        ],
    )
    return kfn(h_split, srcf, dstf, normf)


def _linear_body(x_ref, w_ref, h_ref, *, ns):
    acc = jnp.zeros((MBLK, HALF), _f32)
    for si in range(ns):
        acc = acc + jnp.dot(x_ref[si], w_ref[si], preferred_element_type=_f32)
    h_ref[0] = acc


def _tc_linear(xs, wr, ns):
    grid = (NN // MBLK, NC)
    return pl.pallas_call(
        functools.partial(_linear_body, ns=ns),
        grid=grid,
        in_specs=[
            pl.BlockSpec((ns, MBLK, DIN), lambda m, cc: (0, m, 0)),
            pl.BlockSpec((ns, DIN, HALF), lambda m, cc: (0, 0, cc)),
        ],
        out_specs=pl.BlockSpec((1, MBLK, HALF), lambda m, cc: (cc, m, 0)),
        out_shape=jax.ShapeDtypeStruct((NC, NN, HALF), _f32),
    )(xs, wr)


def _stats_body(agg_ref, h_ref, d2_ref, b_ref, sum_ref, ssq_ref):
    m = pl.program_id(0)
    a = agg_ref[...] + h_ref[...] * d2_ref[...] + b_ref[...]
    ps = jnp.sum(a)
    pss = jnp.sum(a * a)

    @pl.when(m == 0)
    def _():
        sum_ref[0, 0] = ps
        ssq_ref[0, 0] = pss

    @pl.when(m > 0)
    def _():
        sum_ref[0, 0] = sum_ref[0, 0] + ps
        ssq_ref[0, 0] = ssq_ref[0, 0] + pss


def _tc_stats(agg, h, d2col, b3):
    return pl.pallas_call(
        _stats_body,
        grid=(NN // MBLK,),
        in_specs=[
            pl.BlockSpec((NC, MBLK, HALF), lambda m: (0, m, 0)),
            pl.BlockSpec((NC, MBLK, HALF), lambda m: (0, m, 0)),
            pl.BlockSpec((MBLK, 1), lambda m: (m, 0)),
            pl.BlockSpec((NC, 1, HALF), lambda m: (0, 0, 0)),
        ],
        out_specs=(pl.BlockSpec(memory_space=pltpu.SMEM),
                   pl.BlockSpec(memory_space=pltpu.SMEM)),
        out_shape=(jax.ShapeDtypeStruct((1, 1), _f32),
                   jax.ShapeDtypeStruct((1, 1), _f32)),
    )(agg, h, d2col, b3)


def _norm_mm_body(sum_ref, ssq_ref, agg_ref, h_ref, d2_ref, b_ref, w_ref,
                  lb_ref, w2_ref, o_ref):
    cnt = jnp.float32(NN * DMID)
    mu = sum_ref[0, 0] / cnt
    var = ssq_ref[0, 0] / cnt - mu * mu
    a = agg_ref[...] + h_ref[...] * d2_ref[...] + b_ref[...]
    xn = (a - mu) * lax.rsqrt(var + jnp.float32(1e-5))
    y = jnp.maximum(xn * w_ref[...] + lb_ref[...], 0.0)
    w2 = w2_ref[...]
    acc = (jnp.dot(y[0], w2[0], preferred_element_type=_f32)
           + jnp.dot(y[1], w2[1], preferred_element_type=_f32))
    o_ref[0] = acc


def _tc_ln_mm(agg, h, d2col, b3, lnw, lnb, w2r):
    ssum, ssq = _tc_stats(agg, h, d2col, b3)
    return pl.pallas_call(
        _norm_mm_body,
        grid=(NN // MBLK, NC),
        in_specs=[
            pl.BlockSpec(memory_space=pltpu.SMEM),
            pl.BlockSpec(memory_space=pltpu.SMEM),
            pl.BlockSpec((NC, MBLK, HALF), lambda m, cc: (0, m, 0)),
            pl.BlockSpec((NC, MBLK, HALF), lambda m, cc: (0, m, 0)),
            pl.BlockSpec((MBLK, 1), lambda m, cc: (m, 0)),
            pl.BlockSpec((NC, 1, HALF), lambda m, cc: (0, 0, 0)),
            pl.BlockSpec((NC, 1, HALF), lambda m, cc: (0, 0, 0)),
            pl.BlockSpec((NC, 1, HALF), lambda m, cc: (0, 0, 0)),
            pl.BlockSpec((NC, HALF, DMID // 2), lambda m, cc: (0, 0, cc)),
        ],
        out_specs=pl.BlockSpec((1, MBLK, HALF), lambda m, cc: (cc, m, 0)),
        out_shape=jax.ShapeDtypeStruct((NC, NN, HALF), _f32),
    )(ssum, ssq, agg, h, d2col, b3, lnw, lnb, w2r)


def _ln_body_merge(agg_ref, h_ref, d2_ref, b_ref, w_ref, lb_ref, o_ref):
    a = agg_ref[...] + h_ref[...] * d2_ref[...] + b_ref[...]
    mu = jnp.mean(a)
    var = jnp.mean((a - mu) ** 2)
    xn = (a - mu) * lax.rsqrt(var + jnp.float32(1e-5))
    y = jnp.maximum(xn * w_ref[...] + lb_ref[...], 0.0)
    o_ref[:, 0:HALF] = y[0]
    o_ref[:, HALF:DMID] = y[1]


def _tc_ln(agg, h, d2col, b3, lnw, lnb):
    return pl.pallas_call(
        _ln_body_merge,
        grid=(1,),
        in_specs=[
            pl.BlockSpec((NC, NN, HALF), lambda i: (0, 0, 0)),
            pl.BlockSpec((NC, NN, HALF), lambda i: (0, 0, 0)),
            pl.BlockSpec((NN, 1), lambda i: (0, 0)),
            pl.BlockSpec((NC, 1, HALF), lambda i: (0, 0, 0)),
            pl.BlockSpec((NC, 1, HALF), lambda i: (0, 0, 0)),
            pl.BlockSpec((NC, 1, HALF), lambda i: (0, 0, 0)),
        ],
        out_specs=pl.BlockSpec((NN, DMID), lambda i: (0, 0)),
        out_shape=jax.ShapeDtypeStruct((NN, DMID), _f32),
    )(agg, h, d2col, b3, lnw, lnb)


def kernel(x, edge_index, edge_weight, W1, b1, ln1_w, ln1_b, W2, b2, ln2_w, ln2_b):
    src = edge_index[0].astype(_i32)
    dst = edge_index[1].astype(_i32)
    w = edge_weight.astype(_f32)

    norm, dinv2 = _sc_precompute(dst, w, src)
    d2col = dinv2.reshape(NPAD, 1)

    xs1 = x.reshape(1, NN, DIN)
    w1r = W1.reshape(1, DIN, DMID)
    h1 = _tc_linear(xs1, w1r, ns=1)
    agg1 = _sc_aggregate(h1, src, dst, norm)
    w2r = W2.reshape(NC, HALF, DMID)
    h2 = _tc_ln_mm(agg1, h1, d2col, b1.reshape(NC, 1, HALF),
                   ln1_w.reshape(NC, 1, HALF), ln1_b.reshape(NC, 1, HALF),
                   w2r)
    agg2 = _sc_aggregate(h2, src, dst, norm)
    out = _tc_ln(agg2, h2, d2col, b2.reshape(NC, 1, HALF),
                 ln2_w.reshape(NC, 1, HALF), ln2_b.reshape(NC, 1, HALF))
    return out
